# baseline (speedup 1.0000x reference)
import functools
import math

import jax
import jax.numpy as jnp
import numpy as np
from jax import lax
from jax.experimental import pallas as pl
from jax.experimental.pallas import tpu as pltpu
from jax.experimental.pallas import tpu_sc as plsc

_NC = 2
_NS = 16
_CHUNK = 128
_BLK = 16
_LANES = 128

_F32_SIGN = np.uint32(0x80000000)
_BITS = [np.uint32(0x80000000 >> i) for i in range(32)]


def _sc_edge_aggregate(g_pad, src_pad, dst_pad, zero_rows):
    n_pad, d = g_pad.shape
    total_chunks = src_pad.shape[0]
    chunks_per_core = total_chunks // _NC
    n_chunks = chunks_per_core // _NS
    rows_per_tile = n_pad // _NS
    mesh = plsc.VectorSubcoreMesh(core_axis_name="c", subcore_axis_name="s")

    @functools.partial(
        pl.kernel,
        out_type=jax.ShapeDtypeStruct((_NC, n_pad, d), jnp.float32),
        mesh=mesh,
        scratch_types=[
            pltpu.VMEM((_BLK, _CHUNK), jnp.int32),
            pltpu.VMEM((_BLK, _CHUNK), jnp.int32),
            pltpu.VMEM((_CHUNK, d), jnp.float32),
            pltpu.VMEM((_CHUNK, d), jnp.float32),
            pltpu.VMEM_SHARED((n_pad, d), jnp.float32),
            pltpu.SemaphoreType.DMA,
            pltpu.SemaphoreType.DMA,
        ],
    )
    def scatter_kernel(g_hbm, src_hbm, dst_hbm, zero_hbm, out_hbm,
                       src_v, dst_v, rows0_v, rows1_v, acc_sh, sem0, sem1):
        c = lax.axis_index("c")
        s = lax.axis_index("s")
        rows = (rows0_v, rows1_v)
        sems = (sem0, sem1)
        chunk_base = (1 - c) * chunks_per_core + s * n_chunks
        pltpu.sync_copy(zero_hbm, acc_sh.at[pl.ds(s * rows_per_tile, rows_per_tile)])
        plsc.subcore_barrier()

        def gather_start(j, b):
            pltpu.async_copy(g_hbm.at[src_v.at[j]], rows[b], sems[b])

        def gather_wait(j, b):
            pltpu.make_async_copy(g_hbm.at[src_v.at[j]], rows[b], sems[b]).wait()

        def block_body(blk, carry):
            pltpu.sync_copy(src_hbm.at[pl.ds(chunk_base + blk * _BLK, _BLK)], src_v)
            pltpu.sync_copy(dst_hbm.at[pl.ds(chunk_base + blk * _BLK, _BLK)], dst_v)
            gather_start(0, 0)

            def pair_body(i, carry2):
                for b in (0, 1):
                    j = 2 * i + b
                    gather_wait(j, b)

                    @pl.when(j + 1 < _BLK)
                    def _():
                        gather_start(j + 1, 1 - b)

                    pltpu.sync_copy(rows[b], acc_sh.at[dst_v.at[j]], add=True)
                return carry2

            lax.fori_loop(0, _BLK // 2, pair_body, 0)
            return carry

        lax.fori_loop(0, n_chunks // _BLK, block_body, 0)
        plsc.subcore_barrier()
        pltpu.sync_copy(acc_sh.at[pl.ds(s * rows_per_tile, rows_per_tile)],
                        out_hbm.at[c, pl.ds(s * rows_per_tile, rows_per_tile)])

    return scatter_kernel(g_pad, src_pad, dst_pad, zero_rows)


def _dot(a, b):
    return jnp.dot(a, b, preferred_element_type=jnp.float32,
                   precision=lax.Precision.HIGHEST)


def _b2f(b):
    return jnp.where(b, jnp.float32(1), jnp.float32(0))


def _lane_mask(n):
    i0 = lax.broadcasted_iota(jnp.int32, (n, _LANES), 0)
    i1 = lax.broadcasted_iota(jnp.int32, (n, _LANES), 1)
    return _b2f(i1 == i0 % _LANES)


def _col_to_2d(col):
    n = col.shape[0]
    r = n // _LANES
    a = lax.broadcasted_iota(jnp.int32, (r, n), 0)
    i = lax.broadcasted_iota(jnp.int32, (r, n), 1)
    sel = _b2f(i // _LANES == a)
    return _dot(sel, col * _lane_mask(n))


def _2d_to_col(x2d):
    r = x2d.shape[0]
    n = r * _LANES
    i = lax.broadcasted_iota(jnp.int32, (n, r), 0)
    a = lax.broadcasted_iota(jnp.int32, (n, r), 1)
    sel = _b2f(i // _LANES == a)
    cmat = _dot(sel, x2d)
    return jnp.sum(cmat * _lane_mask(n), axis=1, keepdims=True)


def _sortable(s):
    bits = lax.bitcast_convert_type(s, jnp.uint32)
    return jnp.where(bits >= _F32_SIGN, ~bits, bits | _F32_SIGN)


def _masked_kth(key, mask, need):
    t = jnp.uint32(0)
    for bit in _BITS:
        t2 = t | bit
        cnt = jnp.sum(_b2f(mask & (key >= t2)))
        t = jnp.where(cnt >= need, t2, t)
    return t


def _select_topk(score_keys, alive, k):
    r = alive.shape[0]
    row = lax.broadcasted_iota(jnp.int32, (r, _LANES), 0)
    col = lax.broadcasted_iota(jnp.int32, (r, _LANES), 1)
    inv_idx = ~((row * _LANES + col).astype(jnp.uint32))

    eq = alive
    need = jnp.float32(k)
    keep = jnp.zeros_like(alive)
    for key in score_keys:
        t = _masked_kth(key, eq, need)
        gt = eq & (key > t)
        keep = keep | gt
        need = need - jnp.sum(_b2f(gt))
        eq = eq & (key == t)
    t = _masked_kth(inv_idx, eq, need)
    return keep | (eq & (inv_idx >= t))


def _layer_math(part_ref, g_ref, alive_ref, wrel_ref, brel_ref, wroot_ref,
                p_ref, prior_score_refs, k):
    g = g_ref[...]
    n_pad, d = g.shape
    r = n_pad // _LANES
    agg = part_ref[0] + part_ref[1]
    h = jnp.maximum(_dot(agg, wrel_ref[...]) + brel_ref[...]
                    + _dot(g, wroot_ref[...]), 0.0)
    p = p_ref[...]
    pnorm = jnp.sqrt(jnp.sum(p * p))
    s_col = jnp.tanh(_dot(h, p) / pnorm)
    s2d = _col_to_2d(s_col)

    keys = [_sortable(s2d)] + [_sortable(pr[...]) for pr in prior_score_refs]
    keep = _select_topk(keys, alive_ref[...] > 0.5, k)
    kf_col = _2d_to_col(_b2f(keep))

    gp = h * s_col * kf_col
    mx = jnp.max(jnp.where(kf_col > 0.5, gp, -jnp.inf), axis=0, keepdims=True)
    mean = jnp.sum(gp, axis=0, keepdims=True) * jnp.float32(1.0 / k)
    ro = jnp.concatenate([mx, mean], axis=1)
    return gp, _b2f(keep), s2d, ro


def _layer_body(part_ref, g_ref, alive_ref, wrel_ref, brel_ref, wroot_ref,
                p_ref, *rest, k, n_prior):
    prior = rest[:n_prior]
    g_out, alive_out, s_out, ro_out = rest[n_prior:]
    gp, kf, s2d, ro = _layer_math(part_ref, g_ref, alive_ref, wrel_ref,
                                  brel_ref, wroot_ref, p_ref, prior, k)
    g_out[...] = gp
    alive_out[...] = kf
    s_out[...] = s2d
    ro_out[...] = ro


def _final_body(part_ref, g_ref, alive_ref, wrel_ref, brel_ref, wroot_ref,
                p_ref, s1_ref, s2_ref, ro1_ref, ro2_ref, w1_ref, b1_ref,
                w2_ref, b2_ref, w3_ref, b3_ref, out_ref, *, k):
    _, _, _, ro3 = _layer_math(part_ref, g_ref, alive_ref, wrel_ref, brel_ref,
                               wroot_ref, p_ref, (s2_ref, s1_ref), k)
    z = ro1_ref[...] + ro2_ref[...] + ro3
    z = jnp.maximum(_dot(z, w1_ref[...]) + b1_ref[...], 0.0)
    z = jnp.maximum(_dot(z, w2_ref[...]) + b2_ref[...], 0.0)
    z = _dot(z, w3_ref[...]) + b3_ref[...]
    zmax = jnp.max(z, axis=1, keepdims=True)
    out_ref[...] = z - (jnp.log(jnp.sum(jnp.exp(z - zmax), axis=1,
                                        keepdims=True)) + zmax)


def _tc_layer(part, g, alive, wrel, brel, wroot, p_col, priors, k):
    n_pad, d = g.shape
    r = n_pad // _LANES
    return pl.pallas_call(
        functools.partial(_layer_body, k=k, n_prior=len(priors)),
        out_shape=(
            jax.ShapeDtypeStruct((n_pad, d), jnp.float32),
            jax.ShapeDtypeStruct((r, _LANES), jnp.float32),
            jax.ShapeDtypeStruct((r, _LANES), jnp.float32),
            jax.ShapeDtypeStruct((1, 2 * d), jnp.float32),
        ),
    )(part, g, alive, wrel, brel, wroot, p_col, *priors)


def _tc_final(part, g, alive, wrel, brel, wroot, p_col, s1, s2, ro1, ro2,
              w1, b1, w2, b2, w3, b3, k, c):
    return pl.pallas_call(
        functools.partial(_final_body, k=k),
        out_shape=jax.ShapeDtypeStruct((1, c), jnp.float32),
    )(part, g, alive, wrel, brel, wroot, p_col, s1, s2, ro1, ro2,
      w1, b1, w2, b2, w3, b3)


def kernel(x, edge_index, batch, Wrel1, brel1, Wroot1, p1, Wrel2, brel2,
           Wroot2, p2, Wrel3, brel3, Wroot3, p3, W1, b1, W2, b2, W3, b3):
    n, d = x.shape
    e = edge_index.shape[1]
    c_out = b3.shape[0]

    align_n = _NS * _LANES
    n_pad = (n // align_n + 1) * align_n
    r = n_pad // _LANES
    align_e = _NC * _NS * _CHUNK * 2
    e_pad = ((e + align_e - 1) // align_e) * align_e
    pad_id = n_pad - 1

    src = edge_index[0].astype(jnp.int32)
    dst = edge_index[1].astype(jnp.int32)
    if e_pad > e:
        fill = jnp.full((e_pad - e,), pad_id, jnp.int32)
        src = jnp.concatenate([src, fill])
        dst = jnp.concatenate([dst, fill])
    src = src.reshape(e_pad // _CHUNK, _CHUNK)
    dst = dst.reshape(e_pad // _CHUNK, _CHUNK)

    g = jnp.pad(x, ((0, n_pad - n), (0, 0)))
    alive = jnp.reshape(
        jnp.concatenate([jnp.ones((n,), jnp.float32),
                         jnp.zeros((n_pad - n,), jnp.float32)]), (r, _LANES))
    zero_rows = jnp.zeros((n_pad // _NS, d), jnp.float32)

    layers = [(Wrel1, brel1, Wroot1, p1),
              (Wrel2, brel2, Wroot2, p2),
              (Wrel3, brel3, Wroot3, p3)]
    ros = []
    svs = []
    n_alive = n
    for i, (wrel, brel, wroot, p) in enumerate(layers):
        k = int(math.ceil(0.8 * n_alive))
        n_alive = k
        part = _sc_edge_aggregate(g, src, dst, zero_rows)
        brel2d = brel.reshape(1, d)
        p_col = p.reshape(d, 1)
        if i < 2:
            g, alive, sv, ro = _tc_layer(part, g, alive, wrel, brel2d, wroot,
                                         p_col, tuple(reversed(svs)), k)
            ros.append(ro)
            svs.append(sv)
        else:
            out = _tc_final(part, g, alive, wrel, brel2d, wroot, p_col,
                            svs[0], svs[1], ros[0], ros[1], W1,
                            b1.reshape(1, -1), W2, b2.reshape(1, -1),
                            W3, b3.reshape(1, -1), k, c_out)
    return out


# speedup vs baseline: 2.6808x; 2.6808x over previous
import functools
import math

import jax
import jax.numpy as jnp
import numpy as np
from jax import lax
from jax.experimental import pallas as pl
from jax.experimental.pallas import tpu as pltpu
from jax.experimental.pallas import tpu_sc as plsc

_NC = 2
_NS = 16
_CHUNK = 128
_BLK = 16
_LANES = 128

_F32_SIGN = np.uint32(0x80000000)
_BITS = [np.uint32(0x80000000 >> i) for i in range(32)]


def _sc_edge_aggregate(g_pad, src_pad, dst_pad, zero_rows):
    n_pad, d = g_pad.shape
    total_chunks = src_pad.shape[0]
    chunks_per_core = total_chunks // _NC
    n_chunks = chunks_per_core // _NS
    rows_per_tile = n_pad // _NS
    mesh = plsc.VectorSubcoreMesh(core_axis_name="c", subcore_axis_name="s")

    @functools.partial(
        pl.kernel,
        out_type=jax.ShapeDtypeStruct((_NC, n_pad, d), jnp.float32),
        mesh=mesh,
        scratch_types=[
            pltpu.VMEM((_BLK, _CHUNK), jnp.int32),
            pltpu.VMEM((_BLK, _CHUNK), jnp.int32),
            pltpu.VMEM((_CHUNK, d), jnp.float32),
            pltpu.VMEM((_CHUNK, d), jnp.float32),
            pltpu.VMEM_SHARED((n_pad, d), jnp.float32),
            pltpu.SemaphoreType.DMA,
            pltpu.SemaphoreType.DMA,
        ],
    )
    def scatter_kernel(g_hbm, src_hbm, dst_hbm, zero_hbm, out_hbm,
                       src_v, dst_v, rows0_v, rows1_v, acc_sh, sem0, sem1):
        c = lax.axis_index("c")
        s = lax.axis_index("s")
        rows = (rows0_v, rows1_v)
        sems = (sem0, sem1)
        chunk_base = c * chunks_per_core + s * n_chunks
        pltpu.sync_copy(zero_hbm, acc_sh.at[pl.ds(s * rows_per_tile, rows_per_tile)])
        plsc.subcore_barrier()

        def gather_start(j, b):
            pltpu.async_copy(g_hbm.at[src_v.at[j]], rows[b], sems[b])

        def gather_wait(j, b):
            pltpu.make_async_copy(g_hbm.at[src_v.at[j]], rows[b], sems[b]).wait()

        def block_body(blk, carry):
            pltpu.sync_copy(src_hbm.at[pl.ds(chunk_base + blk * _BLK, _BLK)], src_v)
            pltpu.sync_copy(dst_hbm.at[pl.ds(chunk_base + blk * _BLK, _BLK)], dst_v)
            gather_start(0, 0)

            def pair_body(i, carry2):
                for b in (0, 1):
                    j = 2 * i + b
                    gather_wait(j, b)

                    @pl.when(j + 1 < _BLK)
                    def _():
                        gather_start(j + 1, 1 - b)

                    pltpu.sync_copy(rows[b], acc_sh.at[dst_v.at[j]], add=True)
                return carry2

            lax.fori_loop(0, _BLK // 2, pair_body, 0)
            return carry

        lax.fori_loop(0, n_chunks // _BLK, block_body, 0)
        plsc.subcore_barrier()
        pltpu.sync_copy(acc_sh.at[pl.ds(s * rows_per_tile, rows_per_tile)],
                        out_hbm.at[c, pl.ds(s * rows_per_tile, rows_per_tile)])

    return scatter_kernel(g_pad, src_pad, dst_pad, zero_rows)


def _dot(a, b):
    return jnp.dot(a, b, preferred_element_type=jnp.float32,
                   precision=lax.Precision.HIGHEST)


def _b2f(b):
    return jnp.where(b, jnp.float32(1), jnp.float32(0))


def _lane_mask(n):
    i0 = lax.broadcasted_iota(jnp.int32, (n, _LANES), 0)
    i1 = lax.broadcasted_iota(jnp.int32, (n, _LANES), 1)
    return _b2f(i1 == i0 % _LANES)


def _col_to_2d(col):
    n = col.shape[0]
    r = n // _LANES
    a = lax.broadcasted_iota(jnp.int32, (r, n), 0)
    i = lax.broadcasted_iota(jnp.int32, (r, n), 1)
    sel = _b2f(i // _LANES == a)
    return _dot(sel, col * _lane_mask(n))


def _2d_to_col(x2d):
    r = x2d.shape[0]
    n = r * _LANES
    i = lax.broadcasted_iota(jnp.int32, (n, r), 0)
    a = lax.broadcasted_iota(jnp.int32, (n, r), 1)
    sel = _b2f(i // _LANES == a)
    cmat = _dot(sel, x2d)
    return jnp.sum(cmat * _lane_mask(n), axis=1, keepdims=True)


def _sortable(s):
    bits = lax.bitcast_convert_type(s, jnp.uint32)
    return jnp.where(bits >= _F32_SIGN, ~bits, bits | _F32_SIGN)


def _masked_kth(key, mask, need):
    t = jnp.uint32(0)
    for bit in _BITS:
        t2 = t | bit
        cnt = jnp.sum(_b2f(mask & (key >= t2)))
        t = jnp.where(cnt >= need, t2, t)
    return t


def _select_topk(score_keys, alive, k):
    r = alive.shape[0]
    row = lax.broadcasted_iota(jnp.int32, (r, _LANES), 0)
    col = lax.broadcasted_iota(jnp.int32, (r, _LANES), 1)
    inv_idx = ~((row * _LANES + col).astype(jnp.uint32))

    eq = alive
    need = jnp.float32(k)
    keep = jnp.zeros_like(alive)
    for key in score_keys:
        t = _masked_kth(key, eq, need)
        gt = eq & (key > t)
        keep = keep | gt
        need = need - jnp.sum(_b2f(gt))
        eq = eq & (key == t)
    t = _masked_kth(inv_idx, eq, need)
    return keep | (eq & (inv_idx >= t))


def _layer_math(part_ref, g_ref, alive_ref, wrel_ref, brel_ref, wroot_ref,
                p_ref, prior_score_refs, k):
    g = g_ref[...]
    n_pad, d = g.shape
    r = n_pad // _LANES
    agg = part_ref[0] + part_ref[1]
    h = jnp.maximum(_dot(agg, wrel_ref[...]) + brel_ref[...]
                    + _dot(g, wroot_ref[...]), 0.0)
    p = p_ref[...]
    pnorm = jnp.sqrt(jnp.sum(p * p))
    s_col = jnp.tanh(_dot(h, p) / pnorm)
    s2d = _col_to_2d(s_col)

    keys = [_sortable(s2d)] + [_sortable(pr[...]) for pr in prior_score_refs]
    keep = _select_topk(keys, alive_ref[...] > 0.5, k)
    kf_col = _2d_to_col(_b2f(keep))

    gp = h * s_col * kf_col
    mx = jnp.max(jnp.where(kf_col > 0.5, gp, -jnp.inf), axis=0, keepdims=True)
    mean = jnp.sum(gp, axis=0, keepdims=True) * jnp.float32(1.0 / k)
    ro = jnp.concatenate([mx, mean], axis=1)
    return gp, _b2f(keep), s2d, ro


def _layer_body(part_ref, g_ref, alive_ref, wrel_ref, brel_ref, wroot_ref,
                p_ref, *rest, k, n_prior):
    prior = rest[:n_prior]
    g_out, alive_out, s_out, ro_out = rest[n_prior:]
    gp, kf, s2d, ro = _layer_math(part_ref, g_ref, alive_ref, wrel_ref,
                                  brel_ref, wroot_ref, p_ref, prior, k)
    g_out[...] = gp
    alive_out[...] = kf
    s_out[...] = s2d
    ro_out[...] = ro


def _final_body(part_ref, g_ref, alive_ref, wrel_ref, brel_ref, wroot_ref,
                p_ref, s1_ref, s2_ref, ro1_ref, ro2_ref, w1_ref, b1_ref,
                w2_ref, b2_ref, w3_ref, b3_ref, out_ref, *, k):
    _, _, _, ro3 = _layer_math(part_ref, g_ref, alive_ref, wrel_ref, brel_ref,
                               wroot_ref, p_ref, (s2_ref, s1_ref), k)
    z = ro1_ref[...] + ro2_ref[...] + ro3
    z = jnp.maximum(_dot(z, w1_ref[...]) + b1_ref[...], 0.0)
    z = jnp.maximum(_dot(z, w2_ref[...]) + b2_ref[...], 0.0)
    z = _dot(z, w3_ref[...]) + b3_ref[...]
    zmax = jnp.max(z, axis=1, keepdims=True)
    out_ref[...] = z - (jnp.log(jnp.sum(jnp.exp(z - zmax), axis=1,
                                        keepdims=True)) + zmax)


def _tc_layer(part, g, alive, wrel, brel, wroot, p_col, priors, k):
    n_pad, d = g.shape
    r = n_pad // _LANES
    return pl.pallas_call(
        functools.partial(_layer_body, k=k, n_prior=len(priors)),
        out_shape=(
            jax.ShapeDtypeStruct((n_pad, d), jnp.float32),
            jax.ShapeDtypeStruct((r, _LANES), jnp.float32),
            jax.ShapeDtypeStruct((r, _LANES), jnp.float32),
            jax.ShapeDtypeStruct((1, 2 * d), jnp.float32),
        ),
    )(part, g, alive, wrel, brel, wroot, p_col, *priors)


def _tc_final(part, g, alive, wrel, brel, wroot, p_col, s1, s2, ro1, ro2,
              w1, b1, w2, b2, w3, b3, k, c):
    return pl.pallas_call(
        functools.partial(_final_body, k=k),
        out_shape=jax.ShapeDtypeStruct((1, c), jnp.float32),
    )(part, g, alive, wrel, brel, wroot, p_col, s1, s2, ro1, ro2,
      w1, b1, w2, b2, w3, b3)


def kernel(x, edge_index, batch, Wrel1, brel1, Wroot1, p1, Wrel2, brel2,
           Wroot2, p2, Wrel3, brel3, Wroot3, p3, W1, b1, W2, b2, W3, b3):
    n, d = x.shape
    e = edge_index.shape[1]
    c_out = b3.shape[0]

    align_n = _NS * _LANES
    n_pad = (n // align_n + 1) * align_n
    r = n_pad // _LANES
    align_e = _NC * _NS * _CHUNK * 2
    e_pad = ((e + align_e - 1) // align_e) * align_e
    src = edge_index[0].astype(jnp.int32)
    dst = edge_index[1].astype(jnp.int32)
    if e_pad > e:
        fill = n + lax.rem(jnp.arange(e_pad - e, dtype=jnp.int32),
                           jnp.int32(n_pad - n))
        src = jnp.concatenate([src, fill])
        dst = jnp.concatenate([dst, fill])
    src = src.reshape(e_pad // _CHUNK, _CHUNK)
    dst = dst.reshape(e_pad // _CHUNK, _CHUNK)

    g = jnp.pad(x, ((0, n_pad - n), (0, 0)))
    alive = jnp.reshape(
        jnp.concatenate([jnp.ones((n,), jnp.float32),
                         jnp.zeros((n_pad - n,), jnp.float32)]), (r, _LANES))
    zero_rows = jnp.zeros((n_pad // _NS, d), jnp.float32)

    layers = [(Wrel1, brel1, Wroot1, p1),
              (Wrel2, brel2, Wroot2, p2),
              (Wrel3, brel3, Wroot3, p3)]
    ros = []
    svs = []
    n_alive = n
    for i, (wrel, brel, wroot, p) in enumerate(layers):
        k = int(math.ceil(0.8 * n_alive))
        n_alive = k
        part = _sc_edge_aggregate(g, src, dst, zero_rows)
        brel2d = brel.reshape(1, d)
        p_col = p.reshape(d, 1)
        if i < 2:
            g, alive, sv, ro = _tc_layer(part, g, alive, wrel, brel2d, wroot,
                                         p_col, tuple(reversed(svs)), k)
            ros.append(ro)
            svs.append(sv)
        else:
            out = _tc_final(part, g, alive, wrel, brel2d, wroot, p_col,
                            svs[0], svs[1], ros[0], ros[1], W1,
                            b1.reshape(1, -1), W2, b2.reshape(1, -1),
                            W3, b3.reshape(1, -1), k, c_out)
    return out


# speedup vs baseline: 2.9942x; 1.1169x over previous
import functools
import math

import jax
import jax.numpy as jnp
import numpy as np
from jax import lax
from jax.experimental import pallas as pl
from jax.experimental.pallas import tpu as pltpu
from jax.experimental.pallas import tpu_sc as plsc

_NC = 2
_NS = 16
_CHUNK = 128
_BLK = 16
_LANES = 128

_F32_SIGN = np.uint32(0x80000000)
_BITS = [np.uint32(0x80000000 >> i) for i in range(32)]


def _sc_edge_aggregate(g_pad, src_pad, dst_pad, zero_rows):
    n_pad, d = g_pad.shape
    total_chunks = src_pad.shape[0]
    chunks_per_core = total_chunks // _NC
    n_chunks = chunks_per_core // _NS
    rows_per_tile = n_pad // _NS
    mesh = plsc.VectorSubcoreMesh(core_axis_name="c", subcore_axis_name="s")

    @functools.partial(
        pl.kernel,
        out_type=jax.ShapeDtypeStruct((_NC, n_pad, d), jnp.float32),
        mesh=mesh,
        scratch_types=[
            pltpu.VMEM((_BLK, _CHUNK), jnp.int32),
            pltpu.VMEM((_BLK, _CHUNK), jnp.int32),
            pltpu.VMEM((_CHUNK, d), jnp.float32),
            pltpu.VMEM((_CHUNK, d), jnp.float32),
            pltpu.VMEM_SHARED((n_pad, d), jnp.float32),
            pltpu.SemaphoreType.DMA,
            pltpu.SemaphoreType.DMA,
        ],
    )
    def scatter_kernel(g_hbm, src_hbm, dst_hbm, zero_hbm, out_hbm,
                       src_v, dst_v, rows0_v, rows1_v, acc_sh, sem0, sem1):
        c = lax.axis_index("c")
        s = lax.axis_index("s")
        rows = (rows0_v, rows1_v)
        sems = (sem0, sem1)
        chunk_base = c * chunks_per_core + s * n_chunks
        pltpu.sync_copy(zero_hbm, acc_sh.at[pl.ds(s * rows_per_tile, rows_per_tile)])
        plsc.subcore_barrier()

        def gather_start(j, b):
            pltpu.async_copy(g_hbm.at[src_v.at[j]], rows[b], sems[b])

        def gather_wait(j, b):
            pltpu.make_async_copy(g_hbm.at[src_v.at[j]], rows[b], sems[b]).wait()

        def block_body(blk, carry):
            pltpu.sync_copy(src_hbm.at[pl.ds(chunk_base + blk * _BLK, _BLK)], src_v)
            pltpu.sync_copy(dst_hbm.at[pl.ds(chunk_base + blk * _BLK, _BLK)], dst_v)
            gather_start(0, 0)

            def pair_body(i, carry2):
                for b in (0, 1):
                    j = 2 * i + b
                    gather_wait(j, b)

                    @pl.when(j + 1 < _BLK)
                    def _():
                        gather_start(j + 1, 1 - b)

                    pltpu.sync_copy(rows[b], acc_sh.at[dst_v.at[j]], add=True)
                return carry2

            lax.fori_loop(0, _BLK // 2, pair_body, 0)
            return carry

        lax.fori_loop(0, n_chunks // _BLK, block_body, 0)
        plsc.subcore_barrier()
        pltpu.sync_copy(acc_sh.at[pl.ds(s * rows_per_tile, rows_per_tile)],
                        out_hbm.at[c, pl.ds(s * rows_per_tile, rows_per_tile)])

    return scatter_kernel(g_pad, src_pad, dst_pad, zero_rows)


def _dot(a, b):
    return jnp.dot(a, b, preferred_element_type=jnp.float32)


def _dot_exact(a, b):
    return jnp.dot(a, b, preferred_element_type=jnp.float32,
                   precision=lax.Precision.HIGHEST)


def _b2f(b):
    return jnp.where(b, jnp.float32(1), jnp.float32(0))


def _lane_mask(n):
    i0 = lax.broadcasted_iota(jnp.int32, (n, _LANES), 0)
    i1 = lax.broadcasted_iota(jnp.int32, (n, _LANES), 1)
    return _b2f(i1 == i0 % _LANES)


def _col_to_2d(col):
    n = col.shape[0]
    r = n // _LANES
    a = lax.broadcasted_iota(jnp.int32, (r, n), 0)
    i = lax.broadcasted_iota(jnp.int32, (r, n), 1)
    sel = _b2f(i // _LANES == a)
    return _dot_exact(sel, col * _lane_mask(n))


def _2d_to_col(x2d):
    r = x2d.shape[0]
    n = r * _LANES
    i = lax.broadcasted_iota(jnp.int32, (n, r), 0)
    a = lax.broadcasted_iota(jnp.int32, (n, r), 1)
    sel = _b2f(i // _LANES == a)
    cmat = _dot_exact(sel, x2d)
    return jnp.sum(cmat * _lane_mask(n), axis=1, keepdims=True)


def _sortable(s):
    bits = lax.bitcast_convert_type(s, jnp.uint32)
    return jnp.where(bits >= _F32_SIGN, ~bits, bits | _F32_SIGN)


def _masked_kth(key, mask, need):
    t = jnp.uint32(0)
    for bit in _BITS:
        t2 = t | bit
        cnt = jnp.sum(_b2f(mask & (key >= t2)))
        t = jnp.where(cnt >= need, t2, t)
    return t


def _select_topk(score_keys, alive, k):
    r = alive.shape[0]
    row = lax.broadcasted_iota(jnp.int32, (r, _LANES), 0)
    col = lax.broadcasted_iota(jnp.int32, (r, _LANES), 1)
    inv_idx = ~((row * _LANES + col).astype(jnp.uint32))

    eq = alive
    need = jnp.float32(k)
    keep = jnp.zeros_like(alive)
    for key in score_keys:
        t = _masked_kth(key, eq, need)
        gt = eq & (key > t)
        keep = keep | gt
        need = need - jnp.sum(_b2f(gt))
        eq = eq & (key == t)
    t = _masked_kth(inv_idx, eq, need)
    return keep | (eq & (inv_idx >= t))


def _layer_math(part_ref, g_ref, alive_ref, wrel_ref, brel_ref, wroot_ref,
                p_ref, pnorm_ref, prior_score_refs, k):
    g = g_ref[...]
    n_pad, d = g.shape
    r = n_pad // _LANES
    agg = part_ref[0] + part_ref[1]
    h = jnp.maximum(_dot(agg, wrel_ref[...]) + brel_ref[...]
                    + _dot(g, wroot_ref[...]), 0.0)
    p = p_ref[...]
    s_col = jnp.tanh(_dot(h, p) / pnorm_ref[0, 0])
    s2d = _col_to_2d(s_col)

    keys = [_sortable(s2d)] + [_sortable(pr[...]) for pr in prior_score_refs]
    keep = _select_topk(keys, alive_ref[...] > 0.5, k)
    kf_col = _2d_to_col(_b2f(keep))

    gp = h * s_col * kf_col
    mx = jnp.max(jnp.where(kf_col > 0.5, gp, -jnp.inf), axis=0, keepdims=True)
    mean = jnp.sum(gp, axis=0, keepdims=True) / jnp.float32(k)
    ro = jnp.concatenate([mx, mean], axis=1)
    return gp, _b2f(keep), s2d, ro


def _layer_body(part_ref, g_ref, alive_ref, wrel_ref, brel_ref, wroot_ref,
                p_ref, pnorm_ref, *rest, k, n_prior):
    prior = rest[:n_prior]
    g_out, alive_out, s_out, ro_out = rest[n_prior:]
    gp, kf, s2d, ro = _layer_math(part_ref, g_ref, alive_ref, wrel_ref,
                                  brel_ref, wroot_ref, p_ref, pnorm_ref,
                                  prior, k)
    g_out[...] = gp
    alive_out[...] = kf
    s_out[...] = s2d
    ro_out[...] = ro


def _final_body(part_ref, g_ref, alive_ref, wrel_ref, brel_ref, wroot_ref,
                p_ref, pnorm_ref, s1_ref, s2_ref, ro1_ref, ro2_ref, w1_ref,
                b1_ref, w2_ref, b2_ref, w3_ref, b3_ref, out_ref, *, k):
    _, _, _, ro3 = _layer_math(part_ref, g_ref, alive_ref, wrel_ref, brel_ref,
                               wroot_ref, p_ref, pnorm_ref, (s2_ref, s1_ref), k)
    z = ro1_ref[...] + ro2_ref[...] + ro3
    z = jnp.maximum(_dot(z, w1_ref[...]) + b1_ref[...], 0.0)
    z = jnp.maximum(_dot(z, w2_ref[...]) + b2_ref[...], 0.0)
    z = _dot(z, w3_ref[...]) + b3_ref[...]
    shifted = z - jnp.max(z, axis=1, keepdims=True)
    out_ref[...] = shifted - jnp.log(jnp.sum(jnp.exp(shifted), axis=1,
                                             keepdims=True))


def _tc_layer(part, g, alive, wrel, brel, wroot, p_col, pnorm, priors, k):
    n_pad, d = g.shape
    r = n_pad // _LANES
    return pl.pallas_call(
        functools.partial(_layer_body, k=k, n_prior=len(priors)),
        out_shape=(
            jax.ShapeDtypeStruct((n_pad, d), jnp.float32),
            jax.ShapeDtypeStruct((r, _LANES), jnp.float32),
            jax.ShapeDtypeStruct((r, _LANES), jnp.float32),
            jax.ShapeDtypeStruct((1, 2 * d), jnp.float32),
        ),
    )(part, g, alive, wrel, brel, wroot, p_col, pnorm, *priors)


def _tc_final(part, g, alive, wrel, brel, wroot, p_col, pnorm, s1, s2, ro1,
              ro2, w1, b1, w2, b2, w3, b3, k, c):
    return pl.pallas_call(
        functools.partial(_final_body, k=k),
        out_shape=jax.ShapeDtypeStruct((1, c), jnp.float32),
    )(part, g, alive, wrel, brel, wroot, p_col, pnorm, s1, s2, ro1, ro2,
      w1, b1, w2, b2, w3, b3)


def kernel(x, edge_index, batch, Wrel1, brel1, Wroot1, p1, Wrel2, brel2,
           Wroot2, p2, Wrel3, brel3, Wroot3, p3, W1, b1, W2, b2, W3, b3):
    n, d = x.shape
    e = edge_index.shape[1]
    c_out = b3.shape[0]

    align_n = _NS * _LANES
    n_pad = (n // align_n + 1) * align_n
    r = n_pad // _LANES
    align_e = _NC * _NS * _CHUNK * 2
    e_pad = ((e + align_e - 1) // align_e) * align_e
    src = edge_index[0].astype(jnp.int32)
    dst = edge_index[1].astype(jnp.int32)
    if e_pad > e:
        fill = n + lax.rem(jnp.arange(e_pad - e, dtype=jnp.int32),
                           jnp.int32(n_pad - n))
        src = jnp.concatenate([src, fill])
        dst = jnp.concatenate([dst, fill])
    src = src.reshape(e_pad // _CHUNK, _CHUNK)
    dst = dst.reshape(e_pad // _CHUNK, _CHUNK)

    g = jnp.pad(x, ((0, n_pad - n), (0, 0)))
    alive = jnp.reshape(
        jnp.concatenate([jnp.ones((n,), jnp.float32),
                         jnp.zeros((n_pad - n,), jnp.float32)]), (r, _LANES))
    zero_rows = jnp.zeros((n_pad // _NS, d), jnp.float32)

    layers = [(Wrel1, brel1, Wroot1, p1),
              (Wrel2, brel2, Wroot2, p2),
              (Wrel3, brel3, Wroot3, p3)]
    ros = []
    svs = []
    n_alive = n
    for i, (wrel, brel, wroot, p) in enumerate(layers):
        k = int(math.ceil(0.8 * n_alive))
        n_alive = k
        part = _sc_edge_aggregate(g, src, dst, zero_rows)
        brel2d = brel.reshape(1, d)
        p_col = p.reshape(d, 1)
        pnorm = jnp.linalg.norm(p).reshape(1, 1)
        if i < 2:
            g, alive, sv, ro = _tc_layer(part, g, alive, wrel, brel2d, wroot,
                                         p_col, pnorm, tuple(reversed(svs)), k)
            ros.append(ro)
            svs.append(sv)
        else:
            out = _tc_final(part, g, alive, wrel, brel2d, wroot, p_col, pnorm,
                            svs[0], svs[1], ros[0], ros[1], W1,
                            b1.reshape(1, -1), W2, b2.reshape(1, -1),
                            W3, b3.reshape(1, -1), k, c_out)
    return out


# speedup vs baseline: 3.0828x; 1.0296x over previous
import functools
import math

import jax
import jax.numpy as jnp
import numpy as np
from jax import lax
from jax.experimental import pallas as pl
from jax.experimental.pallas import tpu as pltpu
from jax.experimental.pallas import tpu_sc as plsc

_NC = 2
_NS = 16
_CHUNK = 128
_BLK = 40
_LANES = 128

_F32_SIGN = np.uint32(0x80000000)
_BITS = [np.uint32(0x80000000 >> i) for i in range(32)]


def _sc_edge_aggregate(g_pad, src_pad, dst_pad, zero_rows):
    n_pad, d = g_pad.shape
    total_chunks = src_pad.shape[0]
    chunks_per_core = total_chunks // _NC
    n_chunks = chunks_per_core // _NS
    rows_per_tile = n_pad // _NS
    mesh = plsc.VectorSubcoreMesh(core_axis_name="c", subcore_axis_name="s")

    @functools.partial(
        pl.kernel,
        out_type=jax.ShapeDtypeStruct((_NC, n_pad, d), jnp.float32),
        mesh=mesh,
        scratch_types=[
            pltpu.VMEM((_BLK, _CHUNK), jnp.int32),
            pltpu.VMEM((_BLK, _CHUNK), jnp.int32),
            pltpu.VMEM((_CHUNK, d), jnp.float32),
            pltpu.VMEM((_CHUNK, d), jnp.float32),
            pltpu.VMEM_SHARED((n_pad, d), jnp.float32),
            pltpu.SemaphoreType.DMA,
            pltpu.SemaphoreType.DMA,
            pltpu.SemaphoreType.DMA,
            pltpu.SemaphoreType.DMA,
        ],
    )
    def scatter_kernel(g_hbm, src_hbm, dst_hbm, zero_hbm, out_hbm,
                       src_v, dst_v, rows0_v, rows1_v, acc_sh,
                       semg0, semg1, sems0, sems1):
        c = lax.axis_index("c")
        s = lax.axis_index("s")
        rows = (rows0_v, rows1_v)
        sems_g = (semg0, semg1)
        sems_s = (sems0, sems1)
        chunk_base = c * chunks_per_core + s * n_chunks
        pltpu.sync_copy(zero_hbm, acc_sh.at[pl.ds(s * rows_per_tile, rows_per_tile)])
        plsc.subcore_barrier()

        def gather_start(j, b):
            pltpu.async_copy(g_hbm.at[src_v.at[j]], rows[b], sems_g[b])

        def gather_wait(j, b):
            pltpu.make_async_copy(g_hbm.at[src_v.at[j]], rows[b],
                                  sems_g[b]).wait()

        def scatter_start(j, b):
            pltpu.async_copy(rows[b], acc_sh.at[dst_v.at[j]], sems_s[b],
                             add=True)

        def scatter_wait(j, b):
            pltpu.make_async_copy(rows[b], acc_sh.at[dst_v.at[j]],
                                  sems_s[b]).wait()

        def block_body(blk, carry):
            pltpu.sync_copy(src_hbm.at[pl.ds(chunk_base + blk * _BLK, _BLK)], src_v)
            pltpu.sync_copy(dst_hbm.at[pl.ds(chunk_base + blk * _BLK, _BLK)], dst_v)
            gather_start(0, 0)

            def pair_body(i, carry2):
                for b in (0, 1):
                    j = 2 * i + b
                    gather_wait(j, b)
                    scatter_start(j, b)

                    @pl.when(j >= 1)
                    def _():
                        scatter_wait(j - 1, 1 - b)

                    @pl.when(j + 1 < _BLK)
                    def _():
                        gather_start(j + 1, 1 - b)
                return carry2

            lax.fori_loop(0, _BLK // 2, pair_body, 0)
            scatter_wait(_BLK - 1, (_BLK - 1) % 2)
            return carry

        lax.fori_loop(0, n_chunks // _BLK, block_body, 0)
        plsc.subcore_barrier()
        pltpu.sync_copy(acc_sh.at[pl.ds(s * rows_per_tile, rows_per_tile)],
                        out_hbm.at[c, pl.ds(s * rows_per_tile, rows_per_tile)])

    return scatter_kernel(g_pad, src_pad, dst_pad, zero_rows)


def _dot(a, b):
    return jnp.dot(a, b, preferred_element_type=jnp.float32)


def _dot_exact(a, b):
    return jnp.dot(a, b, preferred_element_type=jnp.float32,
                   precision=lax.Precision.HIGHEST)


def _b2f(b):
    return jnp.where(b, jnp.float32(1), jnp.float32(0))


def _lane_mask(n):
    i0 = lax.broadcasted_iota(jnp.int32, (n, _LANES), 0)
    i1 = lax.broadcasted_iota(jnp.int32, (n, _LANES), 1)
    return _b2f(i1 == i0 % _LANES)


def _col_to_2d(col):
    n = col.shape[0]
    r = n // _LANES
    a = lax.broadcasted_iota(jnp.int32, (r, n), 0)
    i = lax.broadcasted_iota(jnp.int32, (r, n), 1)
    sel = _b2f(i // _LANES == a)
    return _dot_exact(sel, col * _lane_mask(n))


def _2d_to_col(x2d):
    r = x2d.shape[0]
    n = r * _LANES
    i = lax.broadcasted_iota(jnp.int32, (n, r), 0)
    a = lax.broadcasted_iota(jnp.int32, (n, r), 1)
    sel = _b2f(i // _LANES == a)
    cmat = _dot_exact(sel, x2d)
    return jnp.sum(cmat * _lane_mask(n), axis=1, keepdims=True)


def _sortable(s):
    bits = lax.bitcast_convert_type(s, jnp.uint32)
    return jnp.where(bits >= _F32_SIGN, ~bits, bits | _F32_SIGN)


def _masked_kth(key, mask, need):
    t = jnp.uint32(0)
    for bit in _BITS:
        t2 = t | bit
        cnt = jnp.sum(_b2f(mask & (key >= t2)))
        t = jnp.where(cnt >= need, t2, t)
    return t


def _select_topk(score_keys, alive, k):
    r = alive.shape[0]
    row = lax.broadcasted_iota(jnp.int32, (r, _LANES), 0)
    col = lax.broadcasted_iota(jnp.int32, (r, _LANES), 1)
    inv_idx = ~((row * _LANES + col).astype(jnp.uint32))

    eq = alive
    need = jnp.float32(k)
    keep = jnp.zeros_like(alive)
    for key in score_keys:
        t = _masked_kth(key, eq, need)
        gt = eq & (key > t)
        keep = keep | gt
        need = need - jnp.sum(_b2f(gt))
        eq = eq & (key == t)
    t = _masked_kth(inv_idx, eq, need)
    return keep | (eq & (inv_idx >= t))


def _layer_math(part_ref, g_ref, alive_ref, wrel_ref, brel_ref, wroot_ref,
                p_ref, pnorm_ref, prior_score_refs, k):
    g = g_ref[...]
    n_pad, d = g.shape
    r = n_pad // _LANES
    agg = part_ref[0] + part_ref[1]
    h = jnp.maximum(_dot(agg, wrel_ref[...]) + brel_ref[...]
                    + _dot(g, wroot_ref[...]), 0.0)
    p = p_ref[...]
    s_col = jnp.tanh(_dot(h, p) / pnorm_ref[0, 0])
    s2d = _col_to_2d(s_col)

    keys = [_sortable(s2d)] + [_sortable(pr[...]) for pr in prior_score_refs]
    keep = _select_topk(keys, alive_ref[...] > 0.5, k)
    kf_col = _2d_to_col(_b2f(keep))

    gp = h * s_col * kf_col
    mx = jnp.max(jnp.where(kf_col > 0.5, gp, -jnp.inf), axis=0, keepdims=True)
    mean = jnp.sum(gp, axis=0, keepdims=True) / jnp.float32(k)
    ro = jnp.concatenate([mx, mean], axis=1)
    return gp, _b2f(keep), s2d, ro


def _layer_body(part_ref, g_ref, alive_ref, wrel_ref, brel_ref, wroot_ref,
                p_ref, pnorm_ref, *rest, k, n_prior):
    prior = rest[:n_prior]
    g_out, alive_out, s_out, ro_out = rest[n_prior:]
    gp, kf, s2d, ro = _layer_math(part_ref, g_ref, alive_ref, wrel_ref,
                                  brel_ref, wroot_ref, p_ref, pnorm_ref,
                                  prior, k)
    g_out[...] = gp
    alive_out[...] = kf
    s_out[...] = s2d
    ro_out[...] = ro


def _final_body(part_ref, g_ref, alive_ref, wrel_ref, brel_ref, wroot_ref,
                p_ref, pnorm_ref, s1_ref, s2_ref, ro1_ref, ro2_ref, w1_ref,
                b1_ref, w2_ref, b2_ref, w3_ref, b3_ref, out_ref, *, k):
    _, _, _, ro3 = _layer_math(part_ref, g_ref, alive_ref, wrel_ref, brel_ref,
                               wroot_ref, p_ref, pnorm_ref, (s2_ref, s1_ref), k)
    z = ro1_ref[...] + ro2_ref[...] + ro3
    z = jnp.maximum(_dot(z, w1_ref[...]) + b1_ref[...], 0.0)
    z = jnp.maximum(_dot(z, w2_ref[...]) + b2_ref[...], 0.0)
    z = _dot(z, w3_ref[...]) + b3_ref[...]
    shifted = z - jnp.max(z, axis=1, keepdims=True)
    out_ref[...] = shifted - jnp.log(jnp.sum(jnp.exp(shifted), axis=1,
                                             keepdims=True))


def _tc_layer(part, g, alive, wrel, brel, wroot, p_col, pnorm, priors, k):
    n_pad, d = g.shape
    r = n_pad // _LANES
    return pl.pallas_call(
        functools.partial(_layer_body, k=k, n_prior=len(priors)),
        out_shape=(
            jax.ShapeDtypeStruct((n_pad, d), jnp.float32),
            jax.ShapeDtypeStruct((r, _LANES), jnp.float32),
            jax.ShapeDtypeStruct((r, _LANES), jnp.float32),
            jax.ShapeDtypeStruct((1, 2 * d), jnp.float32),
        ),
    )(part, g, alive, wrel, brel, wroot, p_col, pnorm, *priors)


def _tc_final(part, g, alive, wrel, brel, wroot, p_col, pnorm, s1, s2, ro1,
              ro2, w1, b1, w2, b2, w3, b3, k, c):
    return pl.pallas_call(
        functools.partial(_final_body, k=k),
        out_shape=jax.ShapeDtypeStruct((1, c), jnp.float32),
    )(part, g, alive, wrel, brel, wroot, p_col, pnorm, s1, s2, ro1, ro2,
      w1, b1, w2, b2, w3, b3)


def kernel(x, edge_index, batch, Wrel1, brel1, Wroot1, p1, Wrel2, brel2,
           Wroot2, p2, Wrel3, brel3, Wroot3, p3, W1, b1, W2, b2, W3, b3):
    n, d = x.shape
    e = edge_index.shape[1]
    c_out = b3.shape[0]

    align_n = _NS * _LANES
    n_pad = (n // align_n + 1) * align_n
    r = n_pad // _LANES
    align_e = _NC * _NS * _CHUNK * 2
    e_pad = ((e + align_e - 1) // align_e) * align_e
    src = edge_index[0].astype(jnp.int32)
    dst = edge_index[1].astype(jnp.int32)
    if e_pad > e:
        fill = n + lax.rem(jnp.arange(e_pad - e, dtype=jnp.int32),
                           jnp.int32(n_pad - n))
        src = jnp.concatenate([src, fill])
        dst = jnp.concatenate([dst, fill])
    src = src.reshape(e_pad // _CHUNK, _CHUNK)
    dst = dst.reshape(e_pad // _CHUNK, _CHUNK)

    g = jnp.pad(x, ((0, n_pad - n), (0, 0)))
    alive = jnp.reshape(
        jnp.concatenate([jnp.ones((n,), jnp.float32),
                         jnp.zeros((n_pad - n,), jnp.float32)]), (r, _LANES))
    zero_rows = jnp.zeros((n_pad // _NS, d), jnp.float32)

    layers = [(Wrel1, brel1, Wroot1, p1),
              (Wrel2, brel2, Wroot2, p2),
              (Wrel3, brel3, Wroot3, p3)]
    ros = []
    svs = []
    n_alive = n
    for i, (wrel, brel, wroot, p) in enumerate(layers):
        k = int(math.ceil(0.8 * n_alive))
        n_alive = k
        part = _sc_edge_aggregate(g, src, dst, zero_rows)
        brel2d = brel.reshape(1, d)
        p_col = p.reshape(d, 1)
        pnorm = jnp.linalg.norm(p).reshape(1, 1)
        if i < 2:
            g, alive, sv, ro = _tc_layer(part, g, alive, wrel, brel2d, wroot,
                                         p_col, pnorm, tuple(reversed(svs)), k)
            ros.append(ro)
            svs.append(sv)
        else:
            out = _tc_final(part, g, alive, wrel, brel2d, wroot, p_col, pnorm,
                            svs[0], svs[1], ros[0], ros[1], W1,
                            b1.reshape(1, -1), W2, b2.reshape(1, -1),
                            W3, b3.reshape(1, -1), k, c_out)
    return out


# speedup vs baseline: 3.1142x; 1.0102x over previous
import functools
import math

import jax
import jax.numpy as jnp
import numpy as np
from jax import lax
from jax.experimental import pallas as pl
from jax.experimental.pallas import tpu as pltpu
from jax.experimental.pallas import tpu_sc as plsc

_NC = 2
_NS = 16
_CHUNK = 128
_BLK = 40
_LANES = 128

_F32_SIGN = np.uint32(0x80000000)
_BITS = [np.uint32(0x80000000 >> i) for i in range(32)]


def _sc_edge_aggregate(g_pad, src_pad, dst_pad, zero_rows):
    n_pad, d = g_pad.shape
    total_chunks = src_pad.shape[0]
    chunks_per_core = total_chunks // _NC
    n_chunks = chunks_per_core // _NS
    rows_per_tile = n_pad // _NS
    mesh = plsc.VectorSubcoreMesh(core_axis_name="c", subcore_axis_name="s")

    @functools.partial(
        pl.kernel,
        out_type=jax.ShapeDtypeStruct((_NC, n_pad, d), jnp.float32),
        mesh=mesh,
        scratch_types=[
            pltpu.VMEM((_BLK, _CHUNK), jnp.int32),
            pltpu.VMEM((_BLK, _CHUNK), jnp.int32),
            pltpu.VMEM((_CHUNK, d), jnp.float32),
            pltpu.VMEM((_CHUNK, d), jnp.float32),
            pltpu.VMEM_SHARED((n_pad, d), jnp.float32),
            pltpu.SemaphoreType.DMA,
            pltpu.SemaphoreType.DMA,
            pltpu.SemaphoreType.DMA,
            pltpu.SemaphoreType.DMA,
        ],
    )
    def scatter_kernel(g_hbm, src_hbm, dst_hbm, zero_hbm, out_hbm,
                       src_v, dst_v, rows0_v, rows1_v, acc_sh,
                       semg0, semg1, sems0, sems1):
        c = lax.axis_index("c")
        s = lax.axis_index("s")
        rows = (rows0_v, rows1_v)
        sems_g = (semg0, semg1)
        sems_s = (sems0, sems1)
        chunk_base = c * chunks_per_core + s * n_chunks
        pltpu.sync_copy(zero_hbm, acc_sh.at[pl.ds(s * rows_per_tile, rows_per_tile)])
        plsc.subcore_barrier()

        def gather_start(j, b):
            pltpu.async_copy(g_hbm.at[src_v.at[j]], rows[b], sems_g[b])

        def gather_wait(j, b):
            pltpu.make_async_copy(g_hbm.at[src_v.at[j]], rows[b],
                                  sems_g[b]).wait()

        def scatter_start(j, b):
            pltpu.async_copy(rows[b], acc_sh.at[dst_v.at[j]], sems_s[b],
                             add=True)

        def scatter_wait(j, b):
            pltpu.make_async_copy(rows[b], acc_sh.at[dst_v.at[j]],
                                  sems_s[b]).wait()

        def block_body(blk, carry):
            pltpu.sync_copy(src_hbm.at[pl.ds(chunk_base + blk * _BLK, _BLK)], src_v)
            pltpu.sync_copy(dst_hbm.at[pl.ds(chunk_base + blk * _BLK, _BLK)], dst_v)
            gather_start(0, 0)

            def pair_body(i, carry2):
                for b in (0, 1):
                    j = 2 * i + b
                    gather_wait(j, b)
                    scatter_start(j, b)

                    @pl.when(j >= 1)
                    def _():
                        scatter_wait(j - 1, 1 - b)

                    @pl.when(j + 1 < _BLK)
                    def _():
                        gather_start(j + 1, 1 - b)
                return carry2

            lax.fori_loop(0, _BLK // 2, pair_body, 0)
            scatter_wait(_BLK - 1, (_BLK - 1) % 2)
            return carry

        lax.fori_loop(0, n_chunks // _BLK, block_body, 0)
        plsc.subcore_barrier()
        pltpu.sync_copy(acc_sh.at[pl.ds(s * rows_per_tile, rows_per_tile)],
                        out_hbm.at[c, pl.ds(s * rows_per_tile, rows_per_tile)])

    return scatter_kernel(g_pad, src_pad, dst_pad, zero_rows)


def _dot(a, b):
    return jnp.dot(a, b, preferred_element_type=jnp.float32)


def _dot_exact(a, b):
    return jnp.dot(a, b, preferred_element_type=jnp.float32,
                   precision=lax.Precision.HIGHEST)


def _b2f(b):
    return jnp.where(b, jnp.float32(1), jnp.float32(0))


def _lane_mask(n):
    i0 = lax.broadcasted_iota(jnp.int32, (n, _LANES), 0)
    i1 = lax.broadcasted_iota(jnp.int32, (n, _LANES), 1)
    return _b2f(i1 == i0 % _LANES)


def _col_to_2d(col):
    n = col.shape[0]
    r = n // _LANES
    a = lax.broadcasted_iota(jnp.int32, (r, n), 0)
    i = lax.broadcasted_iota(jnp.int32, (r, n), 1)
    sel = _b2f(i // _LANES == a)
    return _dot_exact(sel, col * _lane_mask(n))


def _2d_to_col(x2d):
    r = x2d.shape[0]
    n = r * _LANES
    i = lax.broadcasted_iota(jnp.int32, (n, r), 0)
    a = lax.broadcasted_iota(jnp.int32, (n, r), 1)
    sel = _b2f(i // _LANES == a)
    cmat = _dot_exact(sel, x2d)
    return jnp.sum(cmat * _lane_mask(n), axis=1, keepdims=True)


def _sortable(s):
    bits = lax.bitcast_convert_type(s, jnp.uint32)
    return jnp.where(bits >= _F32_SIGN, ~bits, bits | _F32_SIGN)


def _masked_kth(key, mask, need, nbits=32):
    t = jnp.uint32(0)
    for bit in _BITS[32 - nbits:]:
        t2 = t | bit
        cnt = jnp.sum(_b2f(mask & (key >= t2)))
        t = jnp.where(cnt >= need, t2, t)
    return t


def _select_topk(score_keys, alive, k):
    r = alive.shape[0]
    n = r * _LANES
    row = lax.broadcasted_iota(jnp.int32, (r, _LANES), 0)
    col = lax.broadcasted_iota(jnp.int32, (r, _LANES), 1)
    inv_idx = (jnp.int32(n - 1) - (row * _LANES + col)).astype(jnp.uint32)

    eq = alive
    need = jnp.float32(k)
    keep = jnp.zeros_like(alive)
    for key in score_keys:
        t = _masked_kth(key, eq, need)
        gt = eq & (key > t)
        keep = keep | gt
        need = need - jnp.sum(_b2f(gt))
        eq = eq & (key == t)
    t = _masked_kth(inv_idx, eq, need, nbits=(n - 1).bit_length())
    return keep | (eq & (inv_idx >= t))


def _layer_math(part_ref, g_ref, alive_ref, wrel_ref, brel_ref, wroot_ref,
                p_ref, pnorm_ref, prior_score_refs, k):
    g = g_ref[...]
    n_pad, d = g.shape
    r = n_pad // _LANES
    agg = part_ref[0] + part_ref[1]
    h = jnp.maximum(_dot(agg, wrel_ref[...]) + brel_ref[...]
                    + _dot(g, wroot_ref[...]), 0.0)
    p = p_ref[...]
    s_col = jnp.tanh(_dot(h, p) / pnorm_ref[0, 0])
    s2d = _col_to_2d(s_col)

    keys = [_sortable(s2d)] + [_sortable(pr[...]) for pr in prior_score_refs]
    keep = _select_topk(keys, alive_ref[...] > 0.5, k)
    kf_col = _2d_to_col(_b2f(keep))

    gp = h * s_col * kf_col
    mx = jnp.max(jnp.where(kf_col > 0.5, gp, -jnp.inf), axis=0, keepdims=True)
    mean = jnp.sum(gp, axis=0, keepdims=True) / jnp.float32(k)
    ro = jnp.concatenate([mx, mean], axis=1)
    return gp, _b2f(keep), s2d, ro


def _layer_body(part_ref, g_ref, alive_ref, wrel_ref, brel_ref, wroot_ref,
                p_ref, pnorm_ref, *rest, k, n_prior):
    prior = rest[:n_prior]
    g_out, alive_out, s_out, ro_out = rest[n_prior:]
    gp, kf, s2d, ro = _layer_math(part_ref, g_ref, alive_ref, wrel_ref,
                                  brel_ref, wroot_ref, p_ref, pnorm_ref,
                                  prior, k)
    g_out[...] = gp
    alive_out[...] = kf
    s_out[...] = s2d
    ro_out[...] = ro


def _final_body(part_ref, g_ref, alive_ref, wrel_ref, brel_ref, wroot_ref,
                p_ref, pnorm_ref, s1_ref, s2_ref, ro1_ref, ro2_ref, w1_ref,
                b1_ref, w2_ref, b2_ref, w3_ref, b3_ref, out_ref, *, k):
    _, _, _, ro3 = _layer_math(part_ref, g_ref, alive_ref, wrel_ref, brel_ref,
                               wroot_ref, p_ref, pnorm_ref, (s2_ref, s1_ref), k)
    z = ro1_ref[...] + ro2_ref[...] + ro3
    z = jnp.maximum(_dot(z, w1_ref[...]) + b1_ref[...], 0.0)
    z = jnp.maximum(_dot(z, w2_ref[...]) + b2_ref[...], 0.0)
    z = _dot(z, w3_ref[...]) + b3_ref[...]
    shifted = z - jnp.max(z, axis=1, keepdims=True)
    out_ref[...] = shifted - jnp.log(jnp.sum(jnp.exp(shifted), axis=1,
                                             keepdims=True))


def _tc_layer(part, g, alive, wrel, brel, wroot, p_col, pnorm, priors, k):
    n_pad, d = g.shape
    r = n_pad // _LANES
    return pl.pallas_call(
        functools.partial(_layer_body, k=k, n_prior=len(priors)),
        out_shape=(
            jax.ShapeDtypeStruct((n_pad, d), jnp.float32),
            jax.ShapeDtypeStruct((r, _LANES), jnp.float32),
            jax.ShapeDtypeStruct((r, _LANES), jnp.float32),
            jax.ShapeDtypeStruct((1, 2 * d), jnp.float32),
        ),
    )(part, g, alive, wrel, brel, wroot, p_col, pnorm, *priors)


def _tc_final(part, g, alive, wrel, brel, wroot, p_col, pnorm, s1, s2, ro1,
              ro2, w1, b1, w2, b2, w3, b3, k, c):
    return pl.pallas_call(
        functools.partial(_final_body, k=k),
        out_shape=jax.ShapeDtypeStruct((1, c), jnp.float32),
    )(part, g, alive, wrel, brel, wroot, p_col, pnorm, s1, s2, ro1, ro2,
      w1, b1, w2, b2, w3, b3)


def kernel(x, edge_index, batch, Wrel1, brel1, Wroot1, p1, Wrel2, brel2,
           Wroot2, p2, Wrel3, brel3, Wroot3, p3, W1, b1, W2, b2, W3, b3):
    n, d = x.shape
    e = edge_index.shape[1]
    c_out = b3.shape[0]

    align_n = _NS * _LANES
    n_pad = (n // align_n + 1) * align_n
    r = n_pad // _LANES
    align_e = _NC * _NS * _CHUNK * 2
    e_pad = ((e + align_e - 1) // align_e) * align_e
    src = edge_index[0].astype(jnp.int32)
    dst = edge_index[1].astype(jnp.int32)
    if e_pad > e:
        fill = n + lax.rem(jnp.arange(e_pad - e, dtype=jnp.int32),
                           jnp.int32(n_pad - n))
        src = jnp.concatenate([src, fill])
        dst = jnp.concatenate([dst, fill])
    src = src.reshape(e_pad // _CHUNK, _CHUNK)
    dst = dst.reshape(e_pad // _CHUNK, _CHUNK)

    g = jnp.pad(x, ((0, n_pad - n), (0, 0)))
    alive = jnp.reshape(
        jnp.concatenate([jnp.ones((n,), jnp.float32),
                         jnp.zeros((n_pad - n,), jnp.float32)]), (r, _LANES))
    zero_rows = jnp.zeros((n_pad // _NS, d), jnp.float32)

    layers = [(Wrel1, brel1, Wroot1, p1),
              (Wrel2, brel2, Wroot2, p2),
              (Wrel3, brel3, Wroot3, p3)]
    ros = []
    svs = []
    n_alive = n
    for i, (wrel, brel, wroot, p) in enumerate(layers):
        k = int(math.ceil(0.8 * n_alive))
        n_alive = k
        part = _sc_edge_aggregate(g, src, dst, zero_rows)
        brel2d = brel.reshape(1, d)
        p_col = p.reshape(d, 1)
        pnorm = jnp.linalg.norm(p).reshape(1, 1)
        if i < 2:
            g, alive, sv, ro = _tc_layer(part, g, alive, wrel, brel2d, wroot,
                                         p_col, pnorm, tuple(reversed(svs)), k)
            ros.append(ro)
            svs.append(sv)
        else:
            out = _tc_final(part, g, alive, wrel, brel2d, wroot, p_col, pnorm,
                            svs[0], svs[1], ros[0], ros[1], W1,
                            b1.reshape(1, -1), W2, b2.reshape(1, -1),
                            W3, b3.reshape(1, -1), k, c_out)
    return out
